# baseline (speedup 1.0000x reference)
import math

import jax
import jax.numpy as jnp
from jax.experimental import pallas as pl
from jax.experimental.pallas import tpu as pltpu

DIM = 256
ROW_TILE = 4096
QJL_SCALE = math.sqrt(math.pi / 2.0) / DIM


def _tq_kernel(b_ref, c_ref, x_ref, pi_ref, s_ref, o_ref):
    xb = x_ref[...]
    pi = pi_ref[...]
    s = s_ref[...]

    n2 = jnp.sum(xb * xb, axis=1, keepdims=True)
    inv_norms = jax.lax.rsqrt(n2)
    norms = n2 * inv_norms
    x_unit = xb * inv_norms

    y = jax.lax.dot_general(
        x_unit, pi, (((1,), (1,)), ((), ())),
        preferred_element_type=jnp.float32)

    b0 = b_ref[0, 0]
    b1 = b_ref[0, 1]
    b2 = b_ref[0, 2]
    c0 = c_ref[0, 0]
    c1 = c_ref[0, 1]
    c2 = c_ref[0, 2]
    c3 = c_ref[0, 3]
    y_hat = jnp.where(
        y > b1,
        jnp.where(y > b2, c3, c2),
        jnp.where(y > b0, c1, c0),
    )

    x_mse = jax.lax.dot_general(
        y_hat, pi, (((1,), (0,)), ((), ())),
        preferred_element_type=jnp.float32) * norms

    residual = xb - x_mse
    res_norms = jnp.sqrt(jnp.sum(residual * residual, axis=1, keepdims=True))

    projected = jax.lax.dot_general(
        residual, s, (((1,), (1,)), ((), ())),
        preferred_element_type=jnp.float32)
    signs = jnp.where(projected > 0, 1.0, -1.0)

    x_qjl = jax.lax.dot_general(
        signs, s, (((1,), (0,)), ((), ())),
        preferred_element_type=jnp.float32)

    o_ref[...] = x_mse + x_qjl * (QJL_SCALE * res_norms)


def kernel(x, Pi, centroids, boundaries, S):
    n, dim = x.shape
    grid = (n // ROW_TILE,)
    b2d = boundaries.reshape(1, 3)
    c2d = centroids.reshape(1, 4)
    return pl.pallas_call(
        _tq_kernel,
        grid=grid,
        in_specs=[
            pl.BlockSpec(memory_space=pltpu.SMEM),
            pl.BlockSpec(memory_space=pltpu.SMEM),
            pl.BlockSpec((ROW_TILE, dim), lambda i: (i, 0)),
            pl.BlockSpec((dim, dim), lambda i: (0, 0)),
            pl.BlockSpec((dim, dim), lambda i: (0, 0)),
        ],
        out_specs=pl.BlockSpec((ROW_TILE, dim), lambda i: (i, 0)),
        out_shape=jax.ShapeDtypeStruct((n, dim), jnp.float32),
        compiler_params=pltpu.CompilerParams(
            dimension_semantics=("parallel",)),
    )(b2d, c2d, x, Pi, S)
